# baseline (speedup 1.0000x reference)
import functools

import jax
import jax.numpy as jnp
from jax import lax
from jax.experimental import pallas as pl
from jax.experimental.pallas import tpu as pltpu
from jax.experimental.pallas import tpu_sc as plsc

N_TOK = 2048
DIM = 768
NEXP = 8
HID = 4 * DIM
TOK_TILE = 256
VOCAB_TILE = 2048


def _sc_gather(table, idx):
    info = plsc.get_sparse_core_info()
    nw = info.num_cores * info.num_subcores
    n = idx.shape[0]
    d = table.shape[1]
    b_per_w = n // nw
    mesh = plsc.VectorSubcoreMesh(core_axis_name="c", subcore_axis_name="s")

    @functools.partial(
        pl.kernel,
        mesh=mesh,
        out_type=jax.ShapeDtypeStruct((n, d), jnp.float32),
        scratch_types=[
            pltpu.VMEM((b_per_w,), jnp.int32),
            pltpu.VMEM((b_per_w, d), jnp.float32),
            pltpu.SemaphoreType.DMA,
        ],
    )
    def k(table_hbm, idx_hbm, out_hbm, idx_v, rows_v, sem):
        wid = lax.axis_index("s") * info.num_cores + lax.axis_index("c")
        base = wid * b_per_w
        pltpu.sync_copy(idx_hbm.at[pl.ds(base, b_per_w)], idx_v)
        pltpu.async_copy(table_hbm.at[idx_v], rows_v, sem).wait()
        pltpu.sync_copy(rows_v, out_hbm.at[pl.ds(base, b_per_w)])

    return k(table, idx)


def _router_body(h_ref, wg_ref, cmb_ref, aux_ref):
    h = h_ref[...]
    wg = wg_ref[...]
    logits = jnp.dot(h, wg, preferred_element_type=jnp.float32)
    probs = jax.nn.softmax(logits, axis=-1)
    gate = jnp.max(probs, axis=-1, keepdims=True)
    ids = lax.broadcasted_iota(jnp.int32, (N_TOK, NEXP), 1)
    eidx = jnp.min(jnp.where(probs >= gate, ids, NEXP), axis=-1, keepdims=True)
    oh = (ids == eidx).astype(jnp.float32)
    f = jnp.mean(oh, axis=0, keepdims=True)
    p_mean = jnp.mean(probs, axis=0, keepdims=True)
    aux_ref[...] = NEXP * jnp.sum(f * p_mean, axis=1, keepdims=True)
    cmb_ref[...] = oh * gate


def _router(h, wg):
    return pl.pallas_call(
        _router_body,
        out_shape=(
            jax.ShapeDtypeStruct((N_TOK, NEXP), jnp.float32),
            jax.ShapeDtypeStruct((1, 1), jnp.float32),
        ),
    )(h, wg)


def _moe_body(h_ref, cmb_ref, w1_ref, b1_ref, w2_ref, b2_ref, y_ref, acc_ref):
    e = pl.program_id(0)
    t = pl.program_id(1)
    hb = h_ref[...].astype(jnp.bfloat16)
    h1 = jnp.dot(hb, w1_ref[0].astype(jnp.bfloat16),
                 preferred_element_type=jnp.float32) + b1_ref[0]
    a = jax.nn.gelu(h1).astype(jnp.bfloat16)
    eo = jnp.dot(a, w2_ref[0].astype(jnp.bfloat16),
                 preferred_element_type=jnp.float32) + b2_ref[0]
    lane = lax.broadcasted_iota(jnp.int32, (TOK_TILE, NEXP), 1)
    cmb_e = jnp.sum(jnp.where(lane == e, cmb_ref[...], 0.0), axis=1,
                    keepdims=True)
    contrib = eo * cmb_e
    sl = pl.ds(t * TOK_TILE, TOK_TILE)

    @pl.when(e == 0)
    def _():
        acc_ref[sl, :] = contrib

    @pl.when(e > 0)
    def _():
        acc_ref[sl, :] = acc_ref[sl, :] + contrib

    @pl.when(e == NEXP - 1)
    def _():
        y_ref[...] = acc_ref[sl, :]


def _moe(h, cmb, w1, b1, w2, b2):
    nt = N_TOK // TOK_TILE
    return pl.pallas_call(
        _moe_body,
        grid=(NEXP, nt),
        in_specs=[
            pl.BlockSpec((TOK_TILE, DIM), lambda e, t: (t, 0)),
            pl.BlockSpec((TOK_TILE, NEXP), lambda e, t: (t, 0)),
            pl.BlockSpec((1, DIM, HID), lambda e, t: (e, 0, 0)),
            pl.BlockSpec((1, 1, HID), lambda e, t: (e, 0, 0)),
            pl.BlockSpec((1, HID, DIM), lambda e, t: (e, 0, 0)),
            pl.BlockSpec((1, 1, DIM), lambda e, t: (e, 0, 0)),
        ],
        out_specs=pl.BlockSpec((TOK_TILE, DIM), lambda e, t: (t, 0)),
        out_shape=jax.ShapeDtypeStruct((N_TOK, DIM), jnp.float32),
        scratch_shapes=[pltpu.VMEM((N_TOK, DIM), jnp.float32)],
    )(h, cmb, w1.reshape(NEXP, DIM, HID), b1.reshape(NEXP, 1, HID),
      w2.reshape(NEXP, HID, DIM), b2.reshape(NEXP, 1, DIM))


def _proj_body(y_ref, wo_ref, bo_ref, out_ref):
    yb = y_ref[...].astype(jnp.bfloat16)
    wo = wo_ref[...].astype(jnp.bfloat16)
    out_ref[0] = jnp.dot(yb, wo, preferred_element_type=jnp.float32) + bo_ref[...]


def _proj(y, wo, bo2d, vocab):
    nv = pl.cdiv(vocab, VOCAB_TILE)
    nt = N_TOK // TOK_TILE
    return pl.pallas_call(
        _proj_body,
        grid=(nv, nt),
        in_specs=[
            pl.BlockSpec((TOK_TILE, DIM), lambda v, t: (t, 0)),
            pl.BlockSpec((DIM, VOCAB_TILE), lambda v, t: (0, v)),
            pl.BlockSpec((1, VOCAB_TILE), lambda v, t: (0, v)),
        ],
        out_specs=pl.BlockSpec((1, TOK_TILE, VOCAB_TILE), lambda v, t: (0, t, v)),
        out_shape=jax.ShapeDtypeStruct((1, N_TOK, vocab), jnp.float32),
    )(y, wo, bo2d)


def kernel(x, emb, Wg, W1, b1, W2, b2, Wo, bo):
    b, t = x.shape
    vocab = Wo.shape[1]
    idx = x.reshape(-1).astype(jnp.int32)
    h = _sc_gather(emb, idx)
    cmb, aux = _router(h, Wg)
    y = _moe(h, cmb, W1, b1, W2, b2)
    logits = _proj(y, Wo, bo.reshape(1, -1), vocab)
    return logits, aux.reshape(())


# speedup vs baseline: 1.1087x; 1.1087x over previous
import functools

import jax
import jax.numpy as jnp
from jax import lax
from jax.experimental import pallas as pl
from jax.experimental.pallas import tpu as pltpu
from jax.experimental.pallas import tpu_sc as plsc

N_TOK = 2048
DIM = 768
NEXP = 8
HID = 4 * DIM
TOK_TILE = 256
VOCAB_TILE = 2048
PROJ_TOK = 512
PROJ_VOC = 4096


def _sc_gather(table, idx):
    info = plsc.get_sparse_core_info()
    nw = info.num_cores * info.num_subcores
    n = idx.shape[0]
    d = table.shape[1]
    b_per_w = n // nw
    mesh = plsc.VectorSubcoreMesh(core_axis_name="c", subcore_axis_name="s")

    @functools.partial(
        pl.kernel,
        mesh=mesh,
        out_type=jax.ShapeDtypeStruct((n, d), jnp.float32),
        scratch_types=[
            pltpu.VMEM((b_per_w,), jnp.int32),
            pltpu.VMEM((b_per_w, d), jnp.float32),
            pltpu.SemaphoreType.DMA,
        ],
    )
    def k(table_hbm, idx_hbm, out_hbm, idx_v, rows_v, sem):
        wid = lax.axis_index("s") * info.num_cores + lax.axis_index("c")
        base = wid * b_per_w
        pltpu.sync_copy(idx_hbm.at[pl.ds(base, b_per_w)], idx_v)
        pltpu.async_copy(table_hbm.at[idx_v], rows_v, sem).wait()
        pltpu.sync_copy(rows_v, out_hbm.at[pl.ds(base, b_per_w)])

    return k(table, idx)


def _router_body(h_ref, wg_ref, cmb_ref, aux_ref):
    h = h_ref[...]
    wg = wg_ref[...]
    logits = jnp.dot(h, wg, preferred_element_type=jnp.float32)
    probs = jax.nn.softmax(logits, axis=-1)
    gate = jnp.max(probs, axis=-1, keepdims=True)
    ids = lax.broadcasted_iota(jnp.int32, (N_TOK, NEXP), 1)
    eidx = jnp.min(jnp.where(probs >= gate, ids, NEXP), axis=-1, keepdims=True)
    oh = (ids == eidx).astype(jnp.float32)
    f = jnp.mean(oh, axis=0, keepdims=True)
    p_mean = jnp.mean(probs, axis=0, keepdims=True)
    aux_ref[...] = NEXP * jnp.sum(f * p_mean, axis=1, keepdims=True)
    cmb_ref[...] = oh * gate


def _router(h, wg):
    return pl.pallas_call(
        _router_body,
        out_shape=(
            jax.ShapeDtypeStruct((N_TOK, NEXP), jnp.float32),
            jax.ShapeDtypeStruct((1, 1), jnp.float32),
        ),
    )(h, wg)


def _moe_body(h_ref, cmb_ref, w1_ref, b1_ref, w2_ref, b2_ref, y_ref, acc_ref):
    e = pl.program_id(0)
    t = pl.program_id(1)
    hb = h_ref[...].astype(jnp.bfloat16)
    h1 = jnp.dot(hb, w1_ref[0].astype(jnp.bfloat16),
                 preferred_element_type=jnp.float32) + b1_ref[0]
    a = jax.nn.gelu(h1).astype(jnp.bfloat16)
    eo = jnp.dot(a, w2_ref[0].astype(jnp.bfloat16),
                 preferred_element_type=jnp.float32) + b2_ref[0]
    lane = lax.broadcasted_iota(jnp.int32, (TOK_TILE, NEXP), 1)
    cmb_e = jnp.sum(jnp.where(lane == e, cmb_ref[...], 0.0), axis=1,
                    keepdims=True)
    contrib = eo * cmb_e
    sl = pl.ds(t * TOK_TILE, TOK_TILE)

    @pl.when(e == 0)
    def _():
        acc_ref[sl, :] = contrib

    @pl.when(e > 0)
    def _():
        acc_ref[sl, :] = acc_ref[sl, :] + contrib

    @pl.when(e == NEXP - 1)
    def _():
        y_ref[...] = acc_ref[sl, :]


def _moe(h, cmb, w1, b1, w2, b2):
    nt = N_TOK // TOK_TILE
    return pl.pallas_call(
        _moe_body,
        grid=(NEXP, nt),
        in_specs=[
            pl.BlockSpec((TOK_TILE, DIM), lambda e, t: (t, 0)),
            pl.BlockSpec((TOK_TILE, NEXP), lambda e, t: (t, 0)),
            pl.BlockSpec((1, DIM, HID), lambda e, t: (e, 0, 0)),
            pl.BlockSpec((1, 1, HID), lambda e, t: (e, 0, 0)),
            pl.BlockSpec((1, HID, DIM), lambda e, t: (e, 0, 0)),
            pl.BlockSpec((1, 1, DIM), lambda e, t: (e, 0, 0)),
        ],
        out_specs=pl.BlockSpec((TOK_TILE, DIM), lambda e, t: (t, 0)),
        out_shape=jax.ShapeDtypeStruct((N_TOK, DIM), jnp.float32),
        scratch_shapes=[pltpu.VMEM((N_TOK, DIM), jnp.float32)],
    )(h, cmb, w1.reshape(NEXP, DIM, HID), b1.reshape(NEXP, 1, HID),
      w2.reshape(NEXP, HID, DIM), b2.reshape(NEXP, 1, DIM))


def _proj_body(y_ref, wo_ref, bo_ref, out_ref):
    yb = y_ref[...].astype(jnp.bfloat16)
    wo = wo_ref[...].astype(jnp.bfloat16)
    out_ref[0] = jnp.dot(yb, wo, preferred_element_type=jnp.float32) + bo_ref[...]


def _proj(y, wo, bo2d, vocab):
    nv = pl.cdiv(vocab, PROJ_VOC)
    nt = N_TOK // PROJ_TOK
    return pl.pallas_call(
        _proj_body,
        grid=(nv, nt),
        in_specs=[
            pl.BlockSpec((PROJ_TOK, DIM), lambda v, t: (t, 0)),
            pl.BlockSpec((DIM, PROJ_VOC), lambda v, t: (0, v)),
            pl.BlockSpec((1, PROJ_VOC), lambda v, t: (0, v)),
        ],
        out_specs=pl.BlockSpec((1, PROJ_TOK, PROJ_VOC), lambda v, t: (0, t, v)),
        out_shape=jax.ShapeDtypeStruct((1, N_TOK, vocab), jnp.float32),
    )(y, wo, bo2d)


def kernel(x, emb, Wg, W1, b1, W2, b2, Wo, bo):
    b, t = x.shape
    vocab = Wo.shape[1]
    idx = x.reshape(-1).astype(jnp.int32)
    h = _sc_gather(emb, idx)
    cmb, aux = _router(h, Wg)
    y = _moe(h, cmb, W1, b1, W2, b2)
    logits = _proj(y, Wo, bo.reshape(1, -1), vocab)
    return logits, aux.reshape(())


# speedup vs baseline: 1.1100x; 1.0012x over previous
import functools

import jax
import jax.numpy as jnp
from jax import lax
from jax.experimental import pallas as pl
from jax.experimental.pallas import tpu as pltpu
from jax.experimental.pallas import tpu_sc as plsc

N_TOK = 2048
DIM = 768
NEXP = 8
HID = 4 * DIM
TOK_TILE = 256
VOCAB_TILE = 2048
PROJ_TOK = 512
PROJ_VOC = 4096


def _sc_gather(table, idx):
    info = plsc.get_sparse_core_info()
    nw = info.num_cores * info.num_subcores
    n = idx.shape[0]
    d = table.shape[1]
    b_per_w = n // nw
    mesh = plsc.VectorSubcoreMesh(core_axis_name="c", subcore_axis_name="s")

    @functools.partial(
        pl.kernel,
        mesh=mesh,
        out_type=jax.ShapeDtypeStruct((n, d), jnp.float32),
        scratch_types=[
            pltpu.VMEM((b_per_w,), jnp.int32),
            pltpu.VMEM((b_per_w, d), jnp.float32),
            pltpu.SemaphoreType.DMA,
        ],
    )
    def k(table_hbm, idx_hbm, out_hbm, idx_v, rows_v, sem):
        wid = lax.axis_index("s") * info.num_cores + lax.axis_index("c")
        base = wid * b_per_w
        pltpu.sync_copy(idx_hbm.at[pl.ds(base, b_per_w)], idx_v)
        pltpu.async_copy(table_hbm.at[idx_v], rows_v, sem).wait()
        pltpu.sync_copy(rows_v, out_hbm.at[pl.ds(base, b_per_w)])

    return k(table, idx)


def _router_body(h_ref, wg_ref, cmb_ref, aux_ref):
    h = h_ref[...]
    wg = wg_ref[...]
    logits = jnp.dot(h, wg, preferred_element_type=jnp.float32)
    probs = jax.nn.softmax(logits, axis=-1)
    gate = jnp.max(probs, axis=-1, keepdims=True)
    ids = lax.broadcasted_iota(jnp.int32, (N_TOK, NEXP), 1)
    eidx = jnp.min(jnp.where(probs >= gate, ids, NEXP), axis=-1, keepdims=True)
    oh = (ids == eidx).astype(jnp.float32)
    f = jnp.mean(oh, axis=0, keepdims=True)
    p_mean = jnp.mean(probs, axis=0, keepdims=True)
    aux_ref[...] = NEXP * jnp.sum(f * p_mean, axis=1, keepdims=True)
    cmb_ref[...] = oh * gate


def _router(h, wg):
    return pl.pallas_call(
        _router_body,
        out_shape=(
            jax.ShapeDtypeStruct((N_TOK, NEXP), jnp.float32),
            jax.ShapeDtypeStruct((1, 1), jnp.float32),
        ),
    )(h, wg)


def _moe_body(h_ref, cmb_ref, w1_ref, b1_ref, w2_ref, b2_ref, y_ref, acc_ref):
    e = pl.program_id(0)
    t = pl.program_id(1)
    hb = h_ref[...].astype(jnp.bfloat16)
    h1 = jnp.dot(hb, w1_ref[0].astype(jnp.bfloat16),
                 preferred_element_type=jnp.float32) + b1_ref[0]
    a = jax.nn.gelu(h1).astype(jnp.bfloat16)
    eo = jnp.dot(a, w2_ref[0].astype(jnp.bfloat16),
                 preferred_element_type=jnp.float32) + b2_ref[0]
    lane = lax.broadcasted_iota(jnp.int32, (TOK_TILE, NEXP), 1)
    cmb_e = jnp.sum(jnp.where(lane == e, cmb_ref[...], 0.0), axis=1,
                    keepdims=True)
    contrib = eo * cmb_e
    sl = pl.ds(t * TOK_TILE, TOK_TILE)

    @pl.when(e == 0)
    def _():
        acc_ref[sl, :] = contrib

    @pl.when(e > 0)
    def _():
        acc_ref[sl, :] = acc_ref[sl, :] + contrib

    @pl.when(e == NEXP - 1)
    def _():
        y_ref[...] = acc_ref[sl, :]


def _moe(h, cmb, w1, b1, w2, b2):
    nt = N_TOK // TOK_TILE
    return pl.pallas_call(
        _moe_body,
        grid=(NEXP, nt),
        in_specs=[
            pl.BlockSpec((TOK_TILE, DIM), lambda e, t: (t, 0)),
            pl.BlockSpec((TOK_TILE, NEXP), lambda e, t: (t, 0)),
            pl.BlockSpec((1, DIM, HID), lambda e, t: (e, 0, 0)),
            pl.BlockSpec((1, 1, HID), lambda e, t: (e, 0, 0)),
            pl.BlockSpec((1, HID, DIM), lambda e, t: (e, 0, 0)),
            pl.BlockSpec((1, 1, DIM), lambda e, t: (e, 0, 0)),
        ],
        out_specs=pl.BlockSpec((TOK_TILE, DIM), lambda e, t: (t, 0)),
        out_shape=jax.ShapeDtypeStruct((N_TOK, DIM), jnp.float32),
        scratch_shapes=[pltpu.VMEM((N_TOK, DIM), jnp.float32)],
    )(h, cmb, w1.reshape(NEXP, DIM, HID), b1.reshape(NEXP, 1, HID),
      w2.reshape(NEXP, HID, DIM), b2.reshape(NEXP, 1, DIM))


def _proj_body(y_ref, wo_ref, bo_ref, out_ref):
    yb = y_ref[...].astype(jnp.bfloat16)
    wo = wo_ref[...].astype(jnp.bfloat16)
    out_ref[0] = jnp.dot(yb, wo, preferred_element_type=jnp.float32) + bo_ref[...]


def _proj(y, wo, bo2d, vocab):
    nv = pl.cdiv(vocab, PROJ_VOC)
    nt = N_TOK // PROJ_TOK
    return pl.pallas_call(
        _proj_body,
        grid=(nv, nt),
        in_specs=[
            pl.BlockSpec((PROJ_TOK, DIM), lambda v, t: (t, 0)),
            pl.BlockSpec((DIM, PROJ_VOC), lambda v, t: (0, v)),
            pl.BlockSpec((1, PROJ_VOC), lambda v, t: (0, v)),
        ],
        out_specs=pl.BlockSpec((1, PROJ_TOK, PROJ_VOC), lambda v, t: (0, t, v)),
        out_shape=jax.ShapeDtypeStruct((1, N_TOK, vocab), jnp.float32),
        compiler_params=pltpu.CompilerParams(
            dimension_semantics=("parallel", "parallel")),
    )(y, wo, bo2d)


def kernel(x, emb, Wg, W1, b1, W2, b2, Wo, bo):
    b, t = x.shape
    vocab = Wo.shape[1]
    idx = x.reshape(-1).astype(jnp.int32)
    h = _sc_gather(emb, idx)
    cmb, aux = _router(h, Wg)
    y = _moe(h, cmb, W1, b1, W2, b2)
    logits = _proj(y, Wo, bo.reshape(1, -1), vocab)
    return logits, aux.reshape(())


# speedup vs baseline: 1.2071x; 1.0874x over previous
import functools

import jax
import jax.numpy as jnp
from jax import lax
from jax.experimental import pallas as pl
from jax.experimental.pallas import tpu as pltpu
from jax.experimental.pallas import tpu_sc as plsc

N_TOK = 2048
DIM = 768
NEXP = 8
HID = 4 * DIM
TOK_TILE = 256
VOCAB_TILE = 2048
PROJ_TOK = 512
PROJ_VOC = 4096


def _sc_gather(table, idx):
    info = plsc.get_sparse_core_info()
    nw = info.num_cores * info.num_subcores
    n = idx.shape[0]
    d = table.shape[1]
    b_per_w = n // nw
    mesh = plsc.VectorSubcoreMesh(core_axis_name="c", subcore_axis_name="s")

    @functools.partial(
        pl.kernel,
        mesh=mesh,
        out_type=jax.ShapeDtypeStruct((n, d), jnp.float32),
        scratch_types=[
            pltpu.VMEM((b_per_w,), jnp.int32),
            pltpu.VMEM((b_per_w, d), jnp.float32),
            pltpu.SemaphoreType.DMA,
        ],
    )
    def k(table_hbm, idx_hbm, out_hbm, idx_v, rows_v, sem):
        wid = lax.axis_index("s") * info.num_cores + lax.axis_index("c")
        base = wid * b_per_w
        pltpu.sync_copy(idx_hbm.at[pl.ds(base, b_per_w)], idx_v)
        pltpu.async_copy(table_hbm.at[idx_v], rows_v, sem).wait()
        pltpu.sync_copy(rows_v, out_hbm.at[pl.ds(base, b_per_w)])

    return k(table, idx)


def _router_body(h_ref, wg_ref, cmb_ref, aux_ref):
    h = h_ref[...]
    wg = wg_ref[...]
    logits = jnp.dot(h, wg, preferred_element_type=jnp.float32)
    probs = jax.nn.softmax(logits, axis=-1)
    gate = jnp.max(probs, axis=-1, keepdims=True)
    ids = lax.broadcasted_iota(jnp.int32, (N_TOK, NEXP), 1)
    eidx = jnp.min(jnp.where(probs >= gate, ids, NEXP), axis=-1, keepdims=True)
    oh = (ids == eidx).astype(jnp.float32)
    f = jnp.mean(oh, axis=0, keepdims=True)
    p_mean = jnp.mean(probs, axis=0, keepdims=True)
    aux_ref[...] = NEXP * jnp.sum(f * p_mean, axis=1, keepdims=True)
    cmb_ref[...] = oh * gate


def _router(h, wg):
    return pl.pallas_call(
        _router_body,
        out_shape=(
            jax.ShapeDtypeStruct((N_TOK, NEXP), jnp.float32),
            jax.ShapeDtypeStruct((1, 1), jnp.float32),
        ),
    )(h, wg)


def _moe_body(h_ref, cmb_ref, w1_ref, b1_ref, w2_ref, b2_ref, y_ref, acc_ref):
    e = pl.program_id(0)
    t = pl.program_id(1)
    hb = h_ref[...].astype(jnp.bfloat16)
    h1 = jnp.dot(hb, w1_ref[0].astype(jnp.bfloat16),
                 preferred_element_type=jnp.float32) + b1_ref[0]
    a = jax.nn.gelu(h1).astype(jnp.bfloat16)
    eo = jnp.dot(a, w2_ref[0].astype(jnp.bfloat16),
                 preferred_element_type=jnp.float32) + b2_ref[0]
    lane = lax.broadcasted_iota(jnp.int32, (TOK_TILE, NEXP), 1)
    cmb_e = jnp.sum(jnp.where(lane == e, cmb_ref[...], 0.0), axis=1,
                    keepdims=True)
    contrib = eo * cmb_e
    sl = pl.ds(t * TOK_TILE, TOK_TILE)

    @pl.when(e == 0)
    def _():
        acc_ref[sl, :] = contrib

    @pl.when(e > 0)
    def _():
        acc_ref[sl, :] = acc_ref[sl, :] + contrib

    @pl.when(e == NEXP - 1)
    def _():
        y_ref[...] = acc_ref[sl, :]


def _moe(h, cmb, w1, b1, w2, b2):
    nt = N_TOK // TOK_TILE
    return pl.pallas_call(
        _moe_body,
        grid=(NEXP, nt),
        in_specs=[
            pl.BlockSpec((TOK_TILE, DIM), lambda e, t: (t, 0)),
            pl.BlockSpec((TOK_TILE, NEXP), lambda e, t: (t, 0)),
            pl.BlockSpec((1, DIM, HID), lambda e, t: (e, 0, 0)),
            pl.BlockSpec((1, 1, HID), lambda e, t: (e, 0, 0)),
            pl.BlockSpec((1, HID, DIM), lambda e, t: (e, 0, 0)),
            pl.BlockSpec((1, 1, DIM), lambda e, t: (e, 0, 0)),
        ],
        out_specs=pl.BlockSpec((TOK_TILE, DIM), lambda e, t: (t, 0)),
        out_shape=jax.ShapeDtypeStruct((N_TOK, DIM), jnp.float32),
        scratch_shapes=[pltpu.VMEM((N_TOK, DIM), jnp.float32)],
    )(h, cmb, w1.reshape(NEXP, DIM, HID), b1.reshape(NEXP, 1, HID),
      w2.reshape(NEXP, HID, DIM), b2.reshape(NEXP, 1, DIM))


PV = 2048
PT = 512
NCH = N_TOK // PT
NFULL = 48
REM = 1696
RING = 5


def _proj_manual_body(y_ref, wo_hbm, bo_ref, out_hbm,
                      ybf_ref, outr_ref, wor_ref, worem_ref, outrem_ref,
                      out_sem, wo_sem):
    ybf_ref[...] = y_ref[...].astype(jnp.bfloat16)

    def wo_copy(v, slot):
        return pltpu.make_async_copy(
            wo_hbm.at[:, pl.ds(v * PV, PV)], wor_ref.at[slot], wo_sem.at[slot])

    def out_copy(i, slot):
        v = i // NCH
        c = i % NCH
        return pltpu.make_async_copy(
            outr_ref.at[slot],
            out_hbm.at[0, pl.ds(c * PT, PT), pl.ds(v * PV, PV)],
            out_sem.at[slot])

    wo_copy(0, 0).start()

    def vloop(v, _):
        wo_copy(v, lax.rem(v, 2)).wait()

        @pl.when(v + 1 < NFULL)
        def _():
            wo_copy(v + 1, lax.rem(v + 1, 2)).start()

        wob = wor_ref[lax.rem(v, 2)].astype(jnp.bfloat16)
        bt = bo_ref[:, pl.ds(v * PV, PV)]
        for c in range(NCH):
            i = v * NCH + c
            slot = lax.rem(i, RING)

            @pl.when(i >= RING)
            def _():
                out_copy(i - RING, slot).wait()

            res = jnp.dot(ybf_ref[pl.ds(c * PT, PT), :], wob,
                          preferred_element_type=jnp.float32)
            outr_ref[slot] = res + bt
            out_copy(i, slot).start()
        return 0

    lax.fori_loop(0, NFULL, vloop, 0, unroll=False)

    total = NFULL * NCH
    for i in range(total - RING, total):
        out_copy(i, i % RING).wait()

    pltpu.make_async_copy(wo_hbm.at[:, pl.ds(NFULL * PV, REM)], worem_ref,
                          wo_sem.at[0]).start()
    pltpu.make_async_copy(wo_hbm.at[:, pl.ds(NFULL * PV, REM)], worem_ref,
                          wo_sem.at[0]).wait()
    wob = worem_ref[...].astype(jnp.bfloat16)
    bt = bo_ref[:, pl.ds(NFULL * PV, REM)]

    def rem_copy(c):
        return pltpu.make_async_copy(
            outrem_ref.at[c % 2],
            out_hbm.at[0, pl.ds(c * PT, PT), pl.ds(NFULL * PV, REM)],
            out_sem.at[c % 2])

    for c in range(NCH):
        if c >= 2:
            rem_copy(c - 2).wait()
        res = jnp.dot(ybf_ref[pl.ds(c * PT, PT), :], wob,
                      preferred_element_type=jnp.float32)
        outrem_ref[c % 2] = res + bt
        rem_copy(c).start()
    rem_copy(NCH - 2).wait()
    rem_copy(NCH - 1).wait()


def _proj(y, wo, bo2d, vocab):
    return pl.pallas_call(
        _proj_manual_body,
        in_specs=[
            pl.BlockSpec((N_TOK, DIM), lambda: (0, 0)),
            pl.BlockSpec(memory_space=pltpu.HBM),
            pl.BlockSpec((1, vocab), lambda: (0, 0)),
        ],
        out_specs=pl.BlockSpec(memory_space=pltpu.HBM),
        out_shape=jax.ShapeDtypeStruct((1, N_TOK, vocab), jnp.float32),
        scratch_shapes=[
            pltpu.VMEM((N_TOK, DIM), jnp.bfloat16),
            pltpu.VMEM((RING, PT, PV), jnp.float32),
            pltpu.VMEM((2, DIM, PV), jnp.float32),
            pltpu.VMEM((DIM, REM), jnp.float32),
            pltpu.VMEM((2, PT, REM), jnp.float32),
            pltpu.SemaphoreType.DMA((RING,)),
            pltpu.SemaphoreType.DMA((2,)),
        ],
        compiler_params=pltpu.CompilerParams(
            vmem_limit_bytes=100 * 1024 * 1024),
    )(y, wo, bo2d)


def kernel(x, emb, Wg, W1, b1, W2, b2, Wo, bo):
    b, t = x.shape
    vocab = Wo.shape[1]
    idx = x.reshape(-1).astype(jnp.int32)
    h = _sc_gather(emb, idx)
    cmb, aux = _router(h, Wg)
    y = _moe(h, cmb, W1, b1, W2, b2)
    logits = _proj(y, Wo, bo.reshape(1, -1), vocab)
    return logits, aux.reshape(())
